# baseline (speedup 1.0000x reference)
import functools

import jax
import jax.numpy as jnp
from jax import lax
from jax.experimental import pallas as pl
from jax.experimental.pallas import tpu as pltpu
from jax.experimental.pallas import tpu_sc as plsc

VOCAB = 32000
HIDDEN = 768
MAX_POS = 64
LANES = 16
SLICES = HIDDEN // LANES

NUM_CORES = 2
NUM_SUBCORES = 16
NUM_WORKERS = NUM_CORES * NUM_SUBCORES

CHUNK = 32


def _sc_embed(ids_hbm, table_hbm, pos_hbm, out_hbm, pos_v, idx_v, rows_v, sem):
    n_tokens = ids_hbm.shape[0]
    per_worker = n_tokens // NUM_WORKERS
    n_chunks = per_worker // CHUNK

    wid = lax.axis_index("s") * NUM_CORES + lax.axis_index("c")
    base = wid * per_worker

    pltpu.sync_copy(pos_hbm, pos_v)

    def chunk_body(c, carry):
        off = base + c * CHUNK
        pltpu.sync_copy(ids_hbm.at[pl.ds(off, CHUNK)], idx_v)
        pltpu.async_copy(table_hbm.at[idx_v], rows_v, sem).wait()

        pos_half = (c % (MAX_POS // CHUNK)) * CHUNK

        def add_body(t, carry2):
            p = pos_half + t
            for j in range(SLICES):
                sl = pl.ds(j * LANES, LANES)
                rows_v[t, sl] += pos_v[p, sl]
            return carry2

        lax.fori_loop(0, CHUNK, add_body, 0, unroll=False)

        pltpu.sync_copy(rows_v, out_hbm.at[pl.ds(off, CHUNK)])
        return carry

    lax.fori_loop(0, n_chunks, chunk_body, 0, unroll=False)


def kernel(input_ids, token_embedding, position_embedding):
    batch, seq = input_ids.shape
    n_tokens = batch * seq
    ids_flat = input_ids.reshape(n_tokens).astype(jnp.int32)

    mesh = plsc.VectorSubcoreMesh(core_axis_name="c", subcore_axis_name="s")
    run = pl.kernel(
        _sc_embed,
        mesh=mesh,
        out_type=jax.ShapeDtypeStruct((n_tokens, HIDDEN), jnp.float32),
        scratch_types=[
            pltpu.VMEM((MAX_POS, HIDDEN), jnp.float32),
            pltpu.VMEM((CHUNK,), jnp.int32),
            pltpu.VMEM((CHUNK, HIDDEN), jnp.float32),
            pltpu.SemaphoreType.DMA,
        ],
    )
    out = run(ids_flat, token_embedding, position_embedding)
    return out.reshape(batch, seq, HIDDEN)


# speedup vs baseline: 2.0098x; 2.0098x over previous
import functools

import jax
import jax.numpy as jnp
from jax import lax
from jax.experimental import pallas as pl
from jax.experimental.pallas import tpu as pltpu
from jax.experimental.pallas import tpu_sc as plsc

VOCAB = 32000
HIDDEN = 768
MAX_POS = 64
LANES = 16
SLICES = HIDDEN // LANES

NUM_CORES = 2
NUM_SUBCORES = 16
NUM_WORKERS = NUM_CORES * NUM_SUBCORES

CHUNK = 32


def _sc_embed(ids_hbm, table_hbm, pos_hbm, out_hbm, pos_v, idx_v, rows_v, sem):
    n_tokens = ids_hbm.shape[0]
    per_worker = n_tokens // NUM_WORKERS
    n_chunks = per_worker // CHUNK

    wid = lax.axis_index("s") * NUM_CORES + lax.axis_index("c")
    base = wid * per_worker

    pltpu.sync_copy(pos_hbm, pos_v)

    def chunk_body(c, carry):
        off = base + c * CHUNK
        pltpu.sync_copy(ids_hbm.at[pl.ds(off, CHUNK)], idx_v)
        pltpu.async_copy(table_hbm.at[idx_v], rows_v, sem).wait()

        pos_half = (c % (MAX_POS // CHUNK)) * CHUNK

        @plsc.parallel_loop(0, CHUNK, unroll=2)
        def _(t):
            p = pos_half + t
            for j in range(SLICES):
                sl = pl.ds(j * LANES, LANES)
                rows_v[t, sl] += pos_v[p, sl]

        pltpu.sync_copy(rows_v, out_hbm.at[pl.ds(off, CHUNK)])
        return carry

    lax.fori_loop(0, n_chunks, chunk_body, 0, unroll=False)


def kernel(input_ids, token_embedding, position_embedding):
    batch, seq = input_ids.shape
    n_tokens = batch * seq
    ids_flat = input_ids.reshape(n_tokens).astype(jnp.int32)

    mesh = plsc.VectorSubcoreMesh(core_axis_name="c", subcore_axis_name="s")
    run = pl.kernel(
        _sc_embed,
        mesh=mesh,
        out_type=jax.ShapeDtypeStruct((n_tokens, HIDDEN), jnp.float32),
        scratch_types=[
            pltpu.VMEM((MAX_POS, HIDDEN), jnp.float32),
            pltpu.VMEM((CHUNK,), jnp.int32),
            pltpu.VMEM((CHUNK, HIDDEN), jnp.float32),
            pltpu.SemaphoreType.DMA,
        ],
    )
    out = run(ids_flat, token_embedding, position_embedding)
    return out.reshape(batch, seq, HIDDEN)


# speedup vs baseline: 3.5882x; 1.7854x over previous
import jax
import jax.numpy as jnp
from jax import lax
from jax.experimental import pallas as pl
from jax.experimental.pallas import tpu as pltpu
from jax.experimental.pallas import tpu_sc as plsc

VOCAB = 32000
HIDDEN = 768
MAX_POS = 64
LANES = 16
SLICES = HIDDEN // LANES

NUM_CORES = 2
NUM_SUBCORES = 16
NUM_WORKERS = NUM_CORES * NUM_SUBCORES

CHUNK = 32


def _sc_embed(ids_hbm, table_hbm, pos_hbm, out_hbm,
              pos_v, idx0, idx1, rows0, rows1,
              sem_g0, sem_g1, sem_s0, sem_s1):
    n_tokens = ids_hbm.shape[0]
    per_worker = n_tokens // NUM_WORKERS
    n_chunks = per_worker // CHUNK

    wid = lax.axis_index("s") * NUM_CORES + lax.axis_index("c")
    base = wid * per_worker

    idx = (idx0, idx1)
    rows = (rows0, rows1)
    sem_g = (sem_g0, sem_g1)
    sem_s = (sem_s0, sem_s1)

    pltpu.sync_copy(pos_hbm, pos_v)

    pltpu.sync_copy(ids_hbm.at[pl.ds(base, CHUNK)], idx[0])
    pltpu.async_copy(table_hbm.at[idx[0]], rows[0], sem_g[0])

    def step(c, b, o):
        off = base + c * CHUNK

        @pl.when(c + 1 < n_chunks)
        def _():
            pltpu.sync_copy(ids_hbm.at[pl.ds(off + CHUNK, CHUNK)], idx[o])

        @pl.when(c >= 1)
        def _():
            pltpu.make_async_copy(
                rows[o], out_hbm.at[pl.ds(0, CHUNK)], sem_s[o]).wait()

        @pl.when(c + 1 < n_chunks)
        def _():
            pltpu.async_copy(table_hbm.at[idx[o]], rows[o], sem_g[o])

        pltpu.make_async_copy(
            table_hbm.at[idx[b]], rows[b], sem_g[b]).wait()

        pos_half = (c % (MAX_POS // CHUNK)) * CHUNK

        @plsc.parallel_loop(0, CHUNK, unroll=2)
        def _(t):
            p = pos_half + t
            for j in range(SLICES):
                sl = pl.ds(j * LANES, LANES)
                rows[b][t, sl] += pos_v[p, sl]

        pltpu.make_async_copy(
            rows[b], out_hbm.at[pl.ds(off, CHUNK)], sem_s[b]).start()

    def superstep(g, carry):
        step(2 * g, 0, 1)
        step(2 * g + 1, 1, 0)
        return carry

    lax.fori_loop(0, n_chunks // 2, superstep, 0, unroll=False)

    pltpu.make_async_copy(
        rows[1], out_hbm.at[pl.ds(0, CHUNK)], sem_s[1]).wait()


def kernel(input_ids, token_embedding, position_embedding):
    batch, seq = input_ids.shape
    n_tokens = batch * seq
    ids_flat = input_ids.reshape(n_tokens).astype(jnp.int32)

    mesh = plsc.VectorSubcoreMesh(core_axis_name="c", subcore_axis_name="s")
    run = pl.kernel(
        _sc_embed,
        mesh=mesh,
        out_type=jax.ShapeDtypeStruct((n_tokens, HIDDEN), jnp.float32),
        scratch_types=[
            pltpu.VMEM((MAX_POS, HIDDEN), jnp.float32),
            pltpu.VMEM((CHUNK,), jnp.int32),
            pltpu.VMEM((CHUNK,), jnp.int32),
            pltpu.VMEM((CHUNK, HIDDEN), jnp.float32),
            pltpu.VMEM((CHUNK, HIDDEN), jnp.float32),
            pltpu.SemaphoreType.DMA,
            pltpu.SemaphoreType.DMA,
            pltpu.SemaphoreType.DMA,
            pltpu.SemaphoreType.DMA,
        ],
    )
    out = run(ids_flat, token_embedding, position_embedding)
    return out.reshape(batch, seq, HIDDEN)
